# Initial kernel scaffold; baseline (speedup 1.0000x reference)
import jax
import jax.numpy as jnp
from jax.experimental import pallas as pl


def kernel(node_feature, edge_index, edge_type, node_type, params):
    raise NotImplementedError("write your pallas kernel here")


# speedup vs baseline: 1.9368x; 1.9368x over previous
import functools

import jax
import jax.numpy as jnp
from jax import lax
from jax.experimental import pallas as pl
from jax.experimental.pallas import tpu as pltpu
from jax.experimental.pallas import tpu_sc as plsc

N = 10000
D = 128
H = 128
ET = 3
NT = 2
E = 320000

NP = 10240
BN = 1024
NB = NP // BN
ROWS = ET * NP
NQ = 4
QROWS = ROWS // NQ
TRASH = QROWS
SH_ROWS = QROWS + 8
NSC = 2
NSUB = 16
CH = 128
NCH = 160
IB = 40
NIB = NCH // IB
EP = NSUB * NCH * CH
EROWS = EP // CH
ZR = QROWS // NSUB


def _edge_mlp_body(x_ref, w1_ref, b1_ref, w2_ref, b2_ref, out_ref):
    x = x_ref[...]
    h = jnp.maximum(
        jnp.dot(x, w1_ref[0], preferred_element_type=jnp.float32) + b1_ref[0], 0.0)
    o = jnp.dot(h, w2_ref[0], preferred_element_type=jnp.float32) + b2_ref[0]
    out_ref[...] = jnp.maximum(o, 0.0)


def _idx_body(src_ref, dst_ref, et_ref, gidx_ref, sidx_ref):
    src = src_ref[...]
    dst = dst_ref[...]
    et = et_ref[...]
    gidx_ref[...] = et * NP + src
    grow = et * NP + dst
    for q in range(NQ):
        lo = q * QROWS
        inh = (grow >= lo) & (grow < lo + QROWS)
        sidx_ref[q] = jnp.where(inh, grow - lo, TRASH)


def _node_mlp_body(x_ref, a0_ref, a1_ref, a2_ref, nt_ref, w1_ref, b1_ref,
                   w2_ref, b2_ref, out_ref):
    x = x_ref[...]
    enc = jnp.concatenate(
        [jnp.maximum(x, 0.0), a0_ref[...], a1_ref[...], a2_ref[...]], axis=1)
    h0 = jnp.maximum(
        jnp.dot(enc, w1_ref[0], preferred_element_type=jnp.float32) + b1_ref[0], 0.0)
    o0 = jnp.dot(h0, w2_ref[0], preferred_element_type=jnp.float32) + b2_ref[0]
    h1 = jnp.maximum(
        jnp.dot(enc, w1_ref[1], preferred_element_type=jnp.float32) + b1_ref[1], 0.0)
    o1 = jnp.dot(h1, w2_ref[1], preferred_element_type=jnp.float32) + b2_ref[1]
    m = nt_ref[...]
    out_ref[...] = jnp.where(m == 0.0, o0, o1) + x


def _sc_agg_body(m_hbm, gidx_hbm, sidx_hbm, zeros_hbm, out_hbm,
                 agg_sh, gi_v, si_v, bufa, bufb, sema, semb):
    c = lax.axis_index("c")
    s = lax.axis_index("s")
    for p in range(NQ // NSC):
        q = c * (NQ // NSC) + p
        pltpu.sync_copy(zeros_hbm, agg_sh.at[pl.ds(s * ZR, ZR)])
        plsc.subcore_barrier()

        for ib in range(NIB):
            base = s * NCH + ib * IB
            pltpu.sync_copy(gidx_hbm.at[pl.ds(base, IB)], gi_v)
            pltpu.sync_copy(sidx_hbm.at[q].at[pl.ds(base, IB)], si_v)

            @pl.loop(0, IB, step=2)
            def _(i):
                ca = pltpu.async_copy(m_hbm.at[gi_v.at[i]], bufa, sema)
                cb = pltpu.async_copy(m_hbm.at[gi_v.at[i + 1]], bufb, semb)
                ca.wait()
                pltpu.sync_copy(bufa, agg_sh.at[si_v.at[i]], add=True)
                cb.wait()
                pltpu.sync_copy(bufb, agg_sh.at[si_v.at[i + 1]], add=True)

        plsc.subcore_barrier()
        pltpu.sync_copy(agg_sh.at[pl.ds(s * ZR, ZR)],
                        out_hbm.at[pl.ds(q * QROWS + s * ZR, ZR)])


def _sc_agg(m, gidx, sidx, zeros):
    mesh = plsc.VectorSubcoreMesh(core_axis_name="c", subcore_axis_name="s")
    f = pl.kernel(
        _sc_agg_body,
        out_type=jax.ShapeDtypeStruct((ROWS, D), jnp.float32),
        mesh=mesh,
        scratch_types=[
            pltpu.VMEM_SHARED((SH_ROWS, D), jnp.float32),
            pltpu.VMEM((IB, CH), jnp.int32),
            pltpu.VMEM((IB, CH), jnp.int32),
            pltpu.VMEM((CH, D), jnp.float32),
            pltpu.VMEM((CH, D), jnp.float32),
            pltpu.SemaphoreType.DMA,
            pltpu.SemaphoreType.DMA,
        ],
    )
    return f(m, gidx, sidx, zeros)


def kernel(node_feature, edge_index, edge_type, node_type, params):
    f32 = jnp.float32
    w1e = jnp.stack([params["edge"][i][0] for i in range(ET)])
    b1e = jnp.stack([params["edge"][i][1] for i in range(ET)]).reshape(ET, 1, H)
    w2e = jnp.stack([params["edge"][i][2] for i in range(ET)])
    b2e = jnp.stack([params["edge"][i][3] for i in range(ET)]).reshape(ET, 1, D)
    w1n = jnp.stack([params["node"][t][0] for t in range(NT)])
    b1n = jnp.stack([params["node"][t][1] for t in range(NT)]).reshape(NT, 1, H)
    w2n = jnp.stack([params["node"][t][2] for t in range(NT)])
    b2n = jnp.stack([params["node"][t][3] for t in range(NT)]).reshape(NT, 1, D)

    xp = jnp.pad(node_feature, ((0, NP - N), (0, 0)))
    pad = EP - E
    srcp = jnp.concatenate(
        [edge_index[0], jnp.zeros((pad,), jnp.int32)]).reshape(EROWS, CH)
    dstp = jnp.concatenate(
        [edge_index[1], jnp.full((pad,), -1, jnp.int32)]).reshape(EROWS, CH)
    etp = jnp.concatenate(
        [edge_type, jnp.zeros((pad,), jnp.int32)]).reshape(EROWS, CH)

    m = pl.pallas_call(
        _edge_mlp_body,
        grid=(ET, NB),
        in_specs=[
            pl.BlockSpec((BN, D), lambda t, j: (j, 0)),
            pl.BlockSpec((1, D, H), lambda t, j: (t, 0, 0)),
            pl.BlockSpec((1, 1, H), lambda t, j: (t, 0, 0)),
            pl.BlockSpec((1, H, D), lambda t, j: (t, 0, 0)),
            pl.BlockSpec((1, 1, D), lambda t, j: (t, 0, 0)),
        ],
        out_specs=pl.BlockSpec((BN, D), lambda t, j: (t * NB + j, 0)),
        out_shape=jax.ShapeDtypeStruct((ROWS, D), f32),
    )(xp, w1e, b1e, w2e, b2e)

    gidx, sidx = pl.pallas_call(
        _idx_body,
        out_shape=(
            jax.ShapeDtypeStruct((EROWS, CH), jnp.int32),
            jax.ShapeDtypeStruct((NQ, EROWS, CH), jnp.int32),
        ),
    )(srcp, dstp, etp)

    zeros = jnp.zeros((ZR, D), f32)
    agg = _sc_agg(m, gidx, sidx, zeros)

    ntp = jnp.pad(node_type, (0, NP - N)).astype(f32).reshape(NP, 1)
    outp = pl.pallas_call(
        _node_mlp_body,
        grid=(NB,),
        in_specs=[
            pl.BlockSpec((BN, D), lambda j: (j, 0)),
            pl.BlockSpec((BN, D), lambda j: (j, 0)),
            pl.BlockSpec((BN, D), lambda j: (NB + j, 0)),
            pl.BlockSpec((BN, D), lambda j: (2 * NB + j, 0)),
            pl.BlockSpec((BN, 1), lambda j: (j, 0)),
            pl.BlockSpec((NT, D + ET * D, H), lambda j: (0, 0, 0)),
            pl.BlockSpec((NT, 1, H), lambda j: (0, 0, 0)),
            pl.BlockSpec((NT, H, D), lambda j: (0, 0, 0)),
            pl.BlockSpec((NT, 1, D), lambda j: (0, 0, 0)),
        ],
        out_specs=pl.BlockSpec((BN, D), lambda j: (j, 0)),
        out_shape=jax.ShapeDtypeStruct((NP, D), f32),
    )(xp, agg, agg, agg, ntp, w1n, b1n, w2n, b2n)
    return outp[:N]
